# Initial kernel scaffold; baseline (speedup 1.0000x reference)
import jax
import jax.numpy as jnp
from jax.experimental import pallas as pl


def kernel(x, in_degree, out_degree, atom_table, in_table, out_table, graph_token):
    raise NotImplementedError("write your pallas kernel here")


# speedup vs baseline: 8.9375x; 8.9375x over previous
import functools

import jax
import jax.numpy as jnp
from jax import lax
from jax.experimental import pallas as pl
from jax.experimental.pallas import tpu as pltpu
from jax.experimental.pallas import tpu_sc as plsc

_B, _N, _F, _D = 256, 512, 9, 64
_NC, _NS = 2, 16
_NW = _NC * _NS
_GPW = _B // _NW
_C = 64
_CHUNKS = _N // _C
_ROWS_OUT = _N + 1


def _body(deg_max_in, deg_max_out,
          x_ref, ind_ref, outd_ref, atom_ref, int_ref, outt_ref, gt_ref,
          out_ref,
          idxa_v, idxi_v, idxo_v, arows_v, irows_v, orows_v, obuf_v, gt_v,
          sem):
    wid = lax.axis_index("s") * _NC + lax.axis_index("c")
    pltpu.sync_copy(gt_ref, gt_v)

    def graph_body(g, carry):
        b = wid * _GPW + g
        out_base = b * _ROWS_OUT
        pltpu.sync_copy(gt_v, out_ref.at[pl.ds(out_base * _D, _D)])

        def chunk_body(ch, carry2):
            node0 = b * _N + ch * _C
            pltpu.sync_copy(x_ref.at[pl.ds(node0 * _F, _C * _F)], idxa_v)
            pltpu.sync_copy(ind_ref.at[pl.ds(node0, _C)], idxi_v)
            pltpu.sync_copy(outd_ref.at[pl.ds(node0, _C)], idxo_v)
            for j in range(_C // 16):
                s = pl.ds(j * 16, 16)
                idxi_v[s] = jnp.minimum(jnp.maximum(idxi_v[s], 0), deg_max_in)
                idxo_v[s] = jnp.minimum(jnp.maximum(idxo_v[s], 0), deg_max_out)
            cp1 = pltpu.async_copy(atom_ref.at[idxa_v], arows_v, sem)
            cp2 = pltpu.async_copy(int_ref.at[idxi_v], irows_v, sem)
            cp3 = pltpu.async_copy(outt_ref.at[idxo_v], orows_v, sem)
            cp1.wait()
            cp2.wait()
            cp3.wait()

            def node_body(c, carry3):
                for dj in range(_D // 16):
                    s = pl.ds(dj * 16, 16)
                    acc = irows_v[c, s] + orows_v[c, s]
                    for f in range(_F):
                        acc = acc + arows_v[c * _F + f, s]
                    obuf_v[pl.ds(c * _D + dj * 16, 16)] = acc
                return carry3

            lax.fori_loop(0, _C, node_body, 0)
            pltpu.sync_copy(
                obuf_v,
                out_ref.at[pl.ds((out_base + 1 + ch * _C) * _D, _C * _D)])
            return carry2

        lax.fori_loop(0, _CHUNKS, chunk_body, 0)
        return carry

    lax.fori_loop(0, _GPW, graph_body, 0)


@functools.partial(jax.jit, static_argnums=())
def _run(x_flat, ind, outd, atom_table, in_table, out_table, graph_token):
    mesh = plsc.VectorSubcoreMesh(core_axis_name="c", subcore_axis_name="s")
    body = functools.partial(_body, in_table.shape[0] - 1,
                             out_table.shape[0] - 1)
    return pl.kernel(
        body,
        out_type=jax.ShapeDtypeStruct((_B * _ROWS_OUT * _D,), jnp.float32),
        mesh=mesh,
        compiler_params=pltpu.CompilerParams(use_tc_tiling_on_sc=False),
        scratch_types=[
            pltpu.VMEM((_C * _F,), jnp.int32),
            pltpu.VMEM((_C,), jnp.int32),
            pltpu.VMEM((_C,), jnp.int32),
            pltpu.VMEM((_C * _F, _D), jnp.float32),
            pltpu.VMEM((_C, _D), jnp.float32),
            pltpu.VMEM((_C, _D), jnp.float32),
            pltpu.VMEM((_C * _D,), jnp.float32),
            pltpu.VMEM((_D,), jnp.float32),
            pltpu.SemaphoreType.DMA,
        ],
    )(x_flat, ind, outd, atom_table, in_table, out_table, graph_token)


def kernel(x, in_degree, out_degree, atom_table, in_table, out_table,
           graph_token):
    x_flat = x.reshape(-1).astype(jnp.int32)
    ind = in_degree.reshape(-1).astype(jnp.int32)
    outd = out_degree.reshape(-1).astype(jnp.int32)
    out = _run(x_flat, ind, outd, atom_table.astype(jnp.float32),
               in_table.astype(jnp.float32), out_table.astype(jnp.float32),
               graph_token.reshape(-1).astype(jnp.float32))
    return out.reshape(_B, _ROWS_OUT, _D)


# speedup vs baseline: 11.3224x; 1.2668x over previous
import functools

import jax
import jax.numpy as jnp
from jax import lax
from jax.experimental import pallas as pl
from jax.experimental.pallas import tpu as pltpu
from jax.experimental.pallas import tpu_sc as plsc

_B, _N, _F, _D = 256, 512, 9, 64
_NC, _NS = 2, 16
_NW = _NC * _NS
_GPW = _B // _NW
_C = 64
_CHUNKS = _N // _C
_T = _GPW * _CHUNKS
_ROWS_OUT = _N + 1


def _body(deg_max_in, deg_max_out,
          x_ref, ind_ref, outd_ref, atom_ref, int_ref, outt_ref, gt_ref,
          out_ref,
          idxa0, idxi0, idxo0, arows0, irows0, orows0, obuf0,
          idxa1, idxi1, idxo1, arows1, irows1, orows1, obuf1,
          gt_v, sem0, sem1, osem0, osem1):
    wid = lax.axis_index("s") * _NC + lax.axis_index("c")
    node_base = wid * _GPW * _N

    bufs = ((idxa0, idxi0, idxo0, arows0, irows0, orows0, obuf0, sem0, osem0),
            (idxa1, idxi1, idxo1, arows1, irows1, orows1, obuf1, sem1, osem1))

    def fire(t, p):
        idxa, idxi, idxo, arows, irows, orows, _, sem, _ = bufs[p]
        node0 = node_base + t * _C
        pltpu.sync_copy(x_ref.at[pl.ds(node0 * _F, _C * _F)], idxa)
        pltpu.sync_copy(ind_ref.at[pl.ds(node0, _C)], idxi)
        pltpu.sync_copy(outd_ref.at[pl.ds(node0, _C)], idxo)
        for j in range(_C // 16):
            s = pl.ds(j * 16, 16)
            idxi[s] = jnp.minimum(jnp.maximum(idxi[s], 0), deg_max_in)
            idxo[s] = jnp.minimum(jnp.maximum(idxo[s], 0), deg_max_out)
        pltpu.async_copy(atom_ref.at[idxa], arows, sem)
        pltpu.async_copy(int_ref.at[idxi], irows, sem)
        pltpu.async_copy(outt_ref.at[idxo], orows, sem)

    def gather_wait(p):
        idxa, idxi, idxo, arows, irows, orows, _, sem, _ = bufs[p]
        pltpu.make_async_copy(atom_ref.at[idxa], arows, sem).wait()
        pltpu.make_async_copy(int_ref.at[idxi], irows, sem).wait()
        pltpu.make_async_copy(outt_ref.at[idxo], orows, sem).wait()

    def out_slice(t):
        b = wid * _GPW + t // _CHUNKS
        row0 = b * _ROWS_OUT + 1 + (t % _CHUNKS) * _C
        return out_ref.at[pl.ds(row0 * _D, _C * _D)]

    def store_wait(t, p):
        _, _, _, _, _, _, obuf, _, osem = bufs[p]
        pltpu.make_async_copy(obuf, out_slice(t), osem).wait()

    def compute_store(t, p):
        _, _, _, _, irows, orows, obuf, _, osem = bufs[p]
        arows = bufs[p][3]

        def node_body(c, carry):
            for dj in range(_D // 16):
                s = pl.ds(dj * 16, 16)
                acc = irows[c, s] + orows[c, s]
                for f in range(_F):
                    acc = acc + arows[c * _F + f, s]
                obuf[pl.ds(c * _D + dj * 16, 16)] = acc
            return carry

        lax.fori_loop(0, _C, node_body, 0)
        pltpu.async_copy(obuf, out_slice(t), osem)

    pltpu.sync_copy(gt_ref, gt_v)
    for g in range(_GPW):
        b = wid * _GPW + g
        pltpu.sync_copy(gt_v, out_ref.at[pl.ds(b * _ROWS_OUT * _D, _D)])

    fire(0, 0)

    def loop_body(i, carry):
        t0 = 2 * i
        t1 = t0 + 1

        fire(t1, 1)
        gather_wait(0)

        @pl.when(i >= 1)
        def _():
            store_wait(t0 - 2, 0)

        compute_store(t0, 0)

        @pl.when(i < _T // 2 - 1)
        def _():
            fire(t0 + 2, 0)

        gather_wait(1)

        @pl.when(i >= 1)
        def _():
            store_wait(t1 - 2, 1)

        compute_store(t1, 1)
        return carry

    lax.fori_loop(0, _T // 2, loop_body, 0)
    store_wait(_T - 2, 0)
    store_wait(_T - 1, 1)


@jax.jit
def _run(x_flat, ind, outd, atom_table, in_table, out_table, graph_token):
    mesh = plsc.VectorSubcoreMesh(core_axis_name="c", subcore_axis_name="s")
    body = functools.partial(_body, in_table.shape[0] - 1,
                             out_table.shape[0] - 1)
    buf_types = [
        pltpu.VMEM((_C * _F,), jnp.int32),
        pltpu.VMEM((_C,), jnp.int32),
        pltpu.VMEM((_C,), jnp.int32),
        pltpu.VMEM((_C * _F, _D), jnp.float32),
        pltpu.VMEM((_C, _D), jnp.float32),
        pltpu.VMEM((_C, _D), jnp.float32),
        pltpu.VMEM((_C * _D,), jnp.float32),
    ]
    return pl.kernel(
        body,
        out_type=jax.ShapeDtypeStruct((_B * _ROWS_OUT * _D,), jnp.float32),
        mesh=mesh,
        compiler_params=pltpu.CompilerParams(use_tc_tiling_on_sc=False),
        scratch_types=buf_types + buf_types + [
            pltpu.VMEM((_D,), jnp.float32),
            pltpu.SemaphoreType.DMA,
            pltpu.SemaphoreType.DMA,
            pltpu.SemaphoreType.DMA,
            pltpu.SemaphoreType.DMA,
        ],
    )(x_flat, ind, outd, atom_table, in_table, out_table, graph_token)


def kernel(x, in_degree, out_degree, atom_table, in_table, out_table,
           graph_token):
    x_flat = x.reshape(-1).astype(jnp.int32)
    ind = in_degree.reshape(-1).astype(jnp.int32)
    outd = out_degree.reshape(-1).astype(jnp.int32)
    out = _run(x_flat, ind, outd, atom_table.astype(jnp.float32),
               in_table.astype(jnp.float32), out_table.astype(jnp.float32),
               graph_token.reshape(-1).astype(jnp.float32))
    return out.reshape(_B, _ROWS_OUT, _D)


# speedup vs baseline: 11.3311x; 1.0008x over previous
import functools

import jax
import jax.numpy as jnp
from jax import lax
from jax.experimental import pallas as pl
from jax.experimental.pallas import tpu as pltpu
from jax.experimental.pallas import tpu_sc as plsc

_B, _N, _F, _D = 256, 512, 9, 64
_NC, _NS = 2, 16
_NW = _NC * _NS
_GPW = _B // _NW
_C = 64
_CHUNKS = _N // _C
_T = _GPW * _CHUNKS
_ROWS_OUT = _N + 1


def _body(deg_max_in, deg_max_out,
          x_ref, ind_ref, outd_ref, atom_ref, int_ref, outt_ref, gt_ref,
          out_ref,
          idxa0, idxi0, idxo0, arows0, irows0, orows0, obuf0,
          idxa1, idxi1, idxo1, arows1, irows1, orows1, obuf1,
          gt_v, sem0, sem1, osem0, osem1):
    wid = lax.axis_index("s") * _NC + lax.axis_index("c")
    node_base = wid * _GPW * _N

    bufs = ((idxa0, idxi0, idxo0, arows0, irows0, orows0, obuf0, sem0, osem0),
            (idxa1, idxi1, idxo1, arows1, irows1, orows1, obuf1, sem1, osem1))

    def fire(t, p):
        idxa, idxi, idxo, arows, irows, orows, _, sem, _ = bufs[p]
        node0 = node_base + t * _C
        pltpu.sync_copy(x_ref.at[pl.ds(node0 * _F, _C * _F)], idxa)
        pltpu.sync_copy(ind_ref.at[pl.ds(node0, _C)], idxi)
        pltpu.sync_copy(outd_ref.at[pl.ds(node0, _C)], idxo)
        for j in range(_C // 16):
            s = pl.ds(j * 16, 16)
            idxi[s] = jnp.minimum(jnp.maximum(idxi[s], 0), deg_max_in)
            idxo[s] = jnp.minimum(jnp.maximum(idxo[s], 0), deg_max_out)
        pltpu.async_copy(atom_ref.at[idxa], arows, sem)
        pltpu.async_copy(int_ref.at[idxi], irows, sem)
        pltpu.async_copy(outt_ref.at[idxo], orows, sem)

    def gather_wait(p):
        idxa, idxi, idxo, arows, irows, orows, _, sem, _ = bufs[p]
        pltpu.make_async_copy(atom_ref.at[idxa], arows, sem).wait()
        pltpu.make_async_copy(int_ref.at[idxi], irows, sem).wait()
        pltpu.make_async_copy(outt_ref.at[idxo], orows, sem).wait()

    def out_slice(t):
        b = wid * _GPW + t // _CHUNKS
        row0 = 1 + (t % _CHUNKS) * _C
        return out_ref.at[b, pl.ds(row0, _C), :]

    def store_wait(t, p):
        _, _, _, _, _, _, obuf, _, osem = bufs[p]
        pltpu.make_async_copy(obuf, out_slice(t), osem).wait()

    def compute_store(t, p):
        _, _, _, _, irows, orows, obuf, _, osem = bufs[p]
        arows = bufs[p][3]

        def node_body(c2, carry):
            for u in range(2):
                c = 2 * c2 + u
                for dj in range(_D // 16):
                    s = pl.ds(dj * 16, 16)
                    acc = irows[c, s] + orows[c, s]
                    for f in range(_F):
                        acc = acc + arows[c * _F + f, s]
                    obuf[c, s] = acc
            return carry

        lax.fori_loop(0, _C // 2, node_body, 0)
        pltpu.async_copy(obuf, out_slice(t), osem)

    pltpu.sync_copy(gt_ref, gt_v)
    for g in range(_GPW):
        b = wid * _GPW + g
        pltpu.sync_copy(gt_v, out_ref.at[b, pl.ds(0, 1), :])

    fire(0, 0)

    def loop_body(i, carry):
        t0 = 2 * i
        t1 = t0 + 1

        fire(t1, 1)
        gather_wait(0)

        @pl.when(i >= 1)
        def _():
            store_wait(t0 - 2, 0)

        compute_store(t0, 0)

        @pl.when(i < _T // 2 - 1)
        def _():
            fire(t0 + 2, 0)

        gather_wait(1)

        @pl.when(i >= 1)
        def _():
            store_wait(t1 - 2, 1)

        compute_store(t1, 1)
        return carry

    lax.fori_loop(0, _T // 2, loop_body, 0)
    store_wait(_T - 2, 0)
    store_wait(_T - 1, 1)


@jax.jit
def _run(x_flat, ind, outd, atom_table, in_table, out_table, graph_token):
    mesh = plsc.VectorSubcoreMesh(core_axis_name="c", subcore_axis_name="s")
    body = functools.partial(_body, in_table.shape[0] - 1,
                             out_table.shape[0] - 1)
    buf_types = [
        pltpu.VMEM((_C * _F,), jnp.int32),
        pltpu.VMEM((_C,), jnp.int32),
        pltpu.VMEM((_C,), jnp.int32),
        pltpu.VMEM((_C * _F, _D), jnp.float32),
        pltpu.VMEM((_C, _D), jnp.float32),
        pltpu.VMEM((_C, _D), jnp.float32),
        pltpu.VMEM((_C, _D), jnp.float32),
    ]
    return pl.kernel(
        body,
        out_type=jax.ShapeDtypeStruct((_B, _ROWS_OUT, _D), jnp.float32),
        mesh=mesh,
        compiler_params=pltpu.CompilerParams(use_tc_tiling_on_sc=False),
        scratch_types=buf_types + buf_types + [
            pltpu.VMEM((1, _D), jnp.float32),
            pltpu.SemaphoreType.DMA,
            pltpu.SemaphoreType.DMA,
            pltpu.SemaphoreType.DMA,
            pltpu.SemaphoreType.DMA,
        ],
    )(x_flat, ind, outd, atom_table, in_table, out_table, graph_token)


def kernel(x, in_degree, out_degree, atom_table, in_table, out_table,
           graph_token):
    x_flat = x.reshape(-1).astype(jnp.int32)
    ind = in_degree.reshape(-1).astype(jnp.int32)
    outd = out_degree.reshape(-1).astype(jnp.int32)
    return _run(x_flat, ind, outd, atom_table.astype(jnp.float32),
                in_table.astype(jnp.float32), out_table.astype(jnp.float32),
                graph_token.astype(jnp.float32))


# speedup vs baseline: 18.6317x; 1.6443x over previous
import functools

import jax
import jax.numpy as jnp
from jax import lax
from jax.experimental import pallas as pl
from jax.experimental.pallas import tpu as pltpu
from jax.experimental.pallas import tpu_sc as plsc

_B, _N, _F, _D = 256, 512, 9, 64
_BN = _B * _N
_NC, _NS = 2, 16
_NW = _NC * _NS
_GPW = _B // _NW
_C = 64
_CHUNKS = _N // _C
_ROWS_OUT = _N + 1


def _body(deg_max_in, deg_max_out,
          x_ref, ind_ref, outd_ref, atom_ref, int_ref, outt_ref, gt_ref,
          out_ref,
          gx0, gdi0, gdo0, gx1, gdi1, gdo1,
          arows0, irows0, orows0, obuf0,
          arows1, irows1, orows1, obuf1,
          gt_v, ssem0, ssem1, gsem0, gsem1, osem0, osem1):
    wid = lax.axis_index("s") * _NC + lax.axis_index("c")
    graph0 = wid * _GPW

    gx = (gx0, gx1)
    gdi = (gdi0, gdi1)
    gdo = (gdo0, gdo1)
    ssem = (ssem0, ssem1)
    arows = (arows0, arows1)
    irows = (irows0, irows1)
    orows = (orows0, orows1)
    obuf = (obuf0, obuf1)
    gsem = (gsem0, gsem1)
    osem = (osem0, osem1)

    def stage(g, q):
        b = graph0 + g
        for f in range(_F):
            pltpu.async_copy(x_ref.at[pl.ds(f * _BN + b * _N, _N)],
                             gx[q].at[pl.ds(f * _N, _N)], ssem[q])
        pltpu.async_copy(ind_ref.at[pl.ds(b * _N, _N)], gdi[q], ssem[q])
        pltpu.async_copy(outd_ref.at[pl.ds(b * _N, _N)], gdo[q], ssem[q])

    def stage_wait_clip(q):
        for f in range(_F):
            pltpu.make_async_copy(x_ref.at[pl.ds(0, _N)],
                                  gx[q].at[pl.ds(0, _N)], ssem[q]).wait()
        pltpu.make_async_copy(ind_ref.at[pl.ds(0, _N)], gdi[q],
                              ssem[q]).wait()
        pltpu.make_async_copy(outd_ref.at[pl.ds(0, _N)], gdo[q],
                              ssem[q]).wait()
        for j in range(_N // 16):
            s = pl.ds(j * 16, 16)
            gdi[q][s] = jnp.minimum(jnp.maximum(gdi[q][s], 0), deg_max_in)
            gdo[q][s] = jnp.minimum(jnp.maximum(gdo[q][s], 0), deg_max_out)

    def fire(ch, q, p):
        for f in range(_F):
            pltpu.async_copy(
                atom_ref.at[gx[q].at[pl.ds(f * _N + ch * _C, _C)]],
                arows[p].at[pl.ds(f * _C, _C), :], gsem[p])
        pltpu.async_copy(int_ref.at[gdi[q].at[pl.ds(ch * _C, _C)]],
                         irows[p], gsem[p])
        pltpu.async_copy(outt_ref.at[gdo[q].at[pl.ds(ch * _C, _C)]],
                         orows[p], gsem[p])

    def gather_wait(p):
        for f in range(_F):
            pltpu.make_async_copy(atom_ref.at[gx[0].at[pl.ds(0, _C)]],
                                  arows[p].at[pl.ds(f * _C, _C), :],
                                  gsem[p]).wait()
        pltpu.make_async_copy(int_ref.at[gdi[0].at[pl.ds(0, _C)]],
                              irows[p], gsem[p]).wait()
        pltpu.make_async_copy(outt_ref.at[gdo[0].at[pl.ds(0, _C)]],
                              orows[p], gsem[p]).wait()

    def store_drain(p):
        pltpu.make_async_copy(obuf[p], out_ref.at[0, pl.ds(1, _C), :],
                              osem[p]).wait()

    def compute_store(g, ch, p):
        ar, ir, orr, ob = arows[p], irows[p], orows[p], obuf[p]

        def node_body(c, carry):
            for dj in range(_D // 16):
                s = pl.ds(dj * 16, 16)
                acc = ir[c, s] + orr[c, s]
                for f in range(_F):
                    acc = acc + ar[f * _C + c, s]
                ob[c, s] = acc
            return carry

        lax.fori_loop(0, _C, node_body, 0)
        pltpu.async_copy(ob, out_ref.at[graph0 + g, pl.ds(1 + ch * _C, _C), :],
                         osem[p])

    pltpu.sync_copy(gt_ref, gt_v)
    for g in range(_GPW):
        pltpu.sync_copy(gt_v, out_ref.at[graph0 + g, pl.ds(0, 1), :])

    stage(0, 0)
    stage_wait_clip(0)
    stage(1, 1)
    fire(0, 0, 0)

    for g in range(_GPW):
        q = g % 2

        def pair_body(i2, carry, g=g, q=q):
            ch0 = 2 * i2
            fire(ch0 + 1, q, 1)
            gather_wait(0)
            if g == 0:
                @pl.when(i2 >= 1)
                def _():
                    store_drain(0)
            else:
                store_drain(0)
            compute_store(g, ch0, 0)
            fire(ch0 + 2, q, 0)
            gather_wait(1)
            if g == 0:
                @pl.when(i2 >= 1)
                def _():
                    store_drain(1)
            else:
                store_drain(1)
            compute_store(g, ch0 + 1, 1)
            return carry

        lax.fori_loop(0, _CHUNKS // 2 - 1, pair_body, 0)

        fire(_CHUNKS - 1, q, 1)
        gather_wait(0)
        store_drain(0)
        compute_store(g, _CHUNKS - 2, 0)
        if g + 1 < _GPW:
            stage_wait_clip(1 - q)
            fire(0, 1 - q, 0)
        gather_wait(1)
        store_drain(1)
        compute_store(g, _CHUNKS - 1, 1)
        if g + 2 < _GPW:
            stage(g + 2, q)

    store_drain(0)
    store_drain(1)


@jax.jit
def _run(x_fm, ind, outd, atom_table, in_table, out_table, graph_token):
    mesh = plsc.VectorSubcoreMesh(core_axis_name="c", subcore_axis_name="s")
    body = functools.partial(_body, in_table.shape[0] - 1,
                             out_table.shape[0] - 1)
    stage_types = [
        pltpu.VMEM((_F * _N,), jnp.int32),
        pltpu.VMEM((_N,), jnp.int32),
        pltpu.VMEM((_N,), jnp.int32),
    ]
    buf_types = [
        pltpu.VMEM((_F * _C, _D), jnp.float32),
        pltpu.VMEM((_C, _D), jnp.float32),
        pltpu.VMEM((_C, _D), jnp.float32),
        pltpu.VMEM((_C, _D), jnp.float32),
    ]
    return pl.kernel(
        body,
        out_type=jax.ShapeDtypeStruct((_B, _ROWS_OUT, _D), jnp.float32),
        mesh=mesh,
        compiler_params=pltpu.CompilerParams(use_tc_tiling_on_sc=False),
        scratch_types=stage_types + stage_types + buf_types + buf_types + [
            pltpu.VMEM((1, _D), jnp.float32),
            pltpu.SemaphoreType.DMA,
            pltpu.SemaphoreType.DMA,
            pltpu.SemaphoreType.DMA,
            pltpu.SemaphoreType.DMA,
            pltpu.SemaphoreType.DMA,
            pltpu.SemaphoreType.DMA,
        ],
    )(x_fm, ind, outd, atom_table, in_table, out_table, graph_token)


def kernel(x, in_degree, out_degree, atom_table, in_table, out_table,
           graph_token):
    x_fm = x.transpose(2, 0, 1).reshape(-1).astype(jnp.int32)
    ind = in_degree.reshape(-1).astype(jnp.int32)
    outd = out_degree.reshape(-1).astype(jnp.int32)
    return _run(x_fm, ind, outd, atom_table.astype(jnp.float32),
                in_table.astype(jnp.float32), out_table.astype(jnp.float32),
                graph_token.astype(jnp.float32))
